# Initial kernel scaffold; baseline (speedup 1.0000x reference)
import jax
import jax.numpy as jnp
from jax.experimental import pallas as pl


def kernel(x, edge_index, W1, att_src1, att_dst1, b1, lin1_W, lin1_b, W2, att_src2, att_dst2, b2, lin2_W, lin2_b):
    raise NotImplementedError("write your pallas kernel here")


# speedup vs baseline: 17.5248x; 17.5248x over previous
import jax
import jax.numpy as jnp
from jax import lax
from jax.experimental import pallas as pl
from jax.experimental.pallas import tpu as pltpu
from jax.experimental.pallas import tpu_sc as plsc

N = 10000
NP = 10240
E = 320000
C = 128
NW = 32
EPT = E // NW
NCHUNK = 80
K = 128
RPT = NP // 16
R = 1024
G = NP // R
_F32 = jnp.float32
_HI = lax.Precision.HIGHEST


def _attn_bound(ms, md, b_ref):
    zmax = ms[0, 0] + md[0, 0]
    b_ref[...] = jnp.full((1, C), jnp.maximum(zmax, 0.2 * zmax), _F32)


def _tc1_body(x_ref, w1_ref, as_ref, ad_ref, lw_ref,
              h_ref, sk_ref, s_ref, d_ref, b_ref, ms, md):
    i = pl.program_id(0)
    xb = x_ref[...]
    h = jnp.dot(xb, w1_ref[...], precision=_HI, preferred_element_type=_F32)
    h_ref[...] = h
    sk_ref[...] = lax.dot_general(xb, lw_ref[...], (((1,), (1,)), ((), ())),
                                  precision=_HI, preferred_element_type=_F32)
    s = jnp.sum(h * as_ref[...], axis=1)
    d = jnp.sum(h * ad_ref[...], axis=1)
    s_ref[...] = s.reshape(1, 1, R)
    d_ref[...] = d.reshape(1, 1, R)

    @pl.when(i == 0)
    def _():
        ms[0, 0] = -1e30
        md[0, 0] = -1e30

    ms[0, 0] = jnp.maximum(ms[0, 0], jnp.max(s))
    md[0, 0] = jnp.maximum(md[0, 0], jnp.max(d))

    @pl.when(i == pl.num_programs(0) - 1)
    def _():
        _attn_bound(ms, md, b_ref)


def _dense1(x_p, W1, att_src, att_dst, lin_W):
    return pl.pallas_call(
        _tc1_body,
        grid=(G,),
        in_specs=[
            pl.BlockSpec((R, C), lambda i: (i, 0)),
            pl.BlockSpec((C, C), lambda i: (0, 0)),
            pl.BlockSpec((1, C), lambda i: (0, 0)),
            pl.BlockSpec((1, C), lambda i: (0, 0)),
            pl.BlockSpec((C, C), lambda i: (0, 0)),
        ],
        out_specs=[
            pl.BlockSpec((R, C), lambda i: (i, 0)),
            pl.BlockSpec((R, C), lambda i: (i, 0)),
            pl.BlockSpec((1, 1, R), lambda i: (i, 0, 0)),
            pl.BlockSpec((1, 1, R), lambda i: (i, 0, 0)),
            pl.BlockSpec((1, C), lambda i: (0, 0)),
        ],
        out_shape=[
            jax.ShapeDtypeStruct((NP, C), _F32),
            jax.ShapeDtypeStruct((NP, C), _F32),
            jax.ShapeDtypeStruct((G, 1, R), _F32),
            jax.ShapeDtypeStruct((G, 1, R), _F32),
            jax.ShapeDtypeStruct((1, C), _F32),
        ],
        scratch_shapes=[pltpu.SMEM((1, 1), _F32), pltpu.SMEM((1, 1), _F32)],
    )(x_p, W1, att_src, att_dst, lin_W)


def _sca_body(asrc_hbm, adst_hbm, src_hbm, dst_hbm, b_hbm,
              e_hbm, den_hbm,
              asrc_v, adst_v, den_v, b_v,
              srcb0, srcb1, dstb0, dstb1, eb0, eb1,
              sem_s0, sem_s1, sem_d0, sem_d1):
    cc_ = lax.axis_index("c")
    ss_ = lax.axis_index("s")
    wid = cc_ * 16 + ss_

    pltpu.sync_copy(asrc_hbm, asrc_v)
    pltpu.sync_copy(adst_hbm, adst_v)
    pltpu.sync_copy(b_hbm, b_v)

    zv = jnp.zeros((16,), _F32)

    def _zden(i, carry):
        den_v[pl.ds(i * 16, 16)] = zv
        return carry
    lax.fori_loop(0, NP // 16, _zden, 0)

    srcb = (srcb0, srcb1)
    dstb = (dstb0, dstb1)
    eb = (eb0, eb1)
    sem_s = (sem_s0, sem_s1)
    sem_d = (sem_d0, sem_d1)

    def _issue(jj, p):
        pltpu.async_copy(src_hbm.at[wid, jj], srcb[p], sem_s[p])
        pltpu.async_copy(dst_hbm.at[wid, jj], dstb[p], sem_d[p])

    def _wait(jj, p):
        pltpu.make_async_copy(src_hbm.at[wid, jj], srcb[p], sem_s[p]).wait()
        pltpu.make_async_copy(dst_hbm.at[wid, jj], dstb[p], sem_d[p]).wait()

    _issue(0, 0)
    _issue(1, 1)
    bval = b_v[...]

    def _one(jj, p):
        _wait(jj, p)

        def _g(g, carry):
            s16 = srcb[p][pl.ds(g * 16, 16)]
            d16 = dstb[p][pl.ds(g * 16, 16)]
            z = plsc.load_gather(asrc_v, [s16]) + plsc.load_gather(adst_v, [d16])
            z = jnp.maximum(z, 0.2 * z)
            e = jnp.exp(z - bval)
            eb[p][pl.ds(g * 16, 16)] = e
            plsc.addupdate_scatter(den_v, [d16], e)
            return carry
        lax.fori_loop(0, K // 16, _g, 0)
        pltpu.sync_copy(eb[p], e_hbm.at[wid, jj])

        @pl.when(jj + 2 < NCHUNK)
        def _():
            _issue(jnp.minimum(jj + 2, NCHUNK - 1), p)

    def _body(t, carry):
        _one(2 * t, 0)
        _one(2 * t + 1, 1)
        return carry
    lax.fori_loop(0, NCHUNK // 2, _body, 0)

    pltpu.sync_copy(den_v, den_hbm.at[wid])


def _scb_body(h_hbm, src_hbm, dst_hbm, e_hbm,
              agg_hbm,
              rows0, rows1, srcb0, srcb1, dstb0, dstb1, eb0, eb1,
              agg_s,
              sem_r0, sem_r1, sem_s0, sem_s1, sem_d0, sem_d1, sem_e0, sem_e1):
    cc_ = lax.axis_index("c")
    ss_ = lax.axis_index("s")
    wid = cc_ * 16 + ss_

    rows = (rows0, rows1)
    srcb = (srcb0, srcb1)
    dstb = (dstb0, dstb1)
    eb = (eb0, eb1)
    sem_r = (sem_r0, sem_r1)
    sem_s = (sem_s0, sem_s1)
    sem_d = (sem_d0, sem_d1)
    sem_e = (sem_e0, sem_e1)

    zv = jnp.zeros((16,), _F32)

    def _zrow(i, carry):
        for cc in range(C // 16):
            rows0[i, pl.ds(cc * 16, 16)] = zv
        return carry
    lax.fori_loop(0, K, _zrow, 0)
    for t in range(RPT // K):
        pltpu.sync_copy(rows0, agg_s.at[pl.ds(ss_ * RPT + t * K, K)])
    plsc.subcore_barrier()

    def _issue_small(jj, p):
        pltpu.async_copy(src_hbm.at[wid, jj], srcb[p], sem_s[p])
        pltpu.async_copy(dst_hbm.at[wid, jj], dstb[p], sem_d[p])
        pltpu.async_copy(e_hbm.at[wid, jj], eb[p], sem_e[p])

    def _wait_small(jj, p):
        pltpu.make_async_copy(src_hbm.at[wid, jj], srcb[p], sem_s[p]).wait()
        pltpu.make_async_copy(dst_hbm.at[wid, jj], dstb[p], sem_d[p]).wait()
        pltpu.make_async_copy(e_hbm.at[wid, jj], eb[p], sem_e[p]).wait()

    def _issue_rows(jj, p):
        pltpu.async_copy(h_hbm.at[srcb[p]], rows[p], sem_r[p])

    def _wait_rows(jj, p):
        pltpu.make_async_copy(h_hbm.at[srcb[p]], rows[p], sem_r[p]).wait()

    _issue_small(0, 0)
    _issue_small(1, 1)
    _wait_small(0, 0)
    _issue_rows(0, 0)

    def _one(jj, p):
        q = 1 - p
        @pl.when(jj + 1 < NCHUNK)
        def _():
            jn = jnp.minimum(jj + 1, NCHUNK - 1)
            _wait_small(jn, q)
            _issue_rows(jn, q)

        _wait_rows(jj, p)

        def _k(k, carry):
            w = plsc.load_gather(eb[p], [jnp.full((16,), k, jnp.int32)])
            for cc in range(C // 16):
                sl = pl.ds(cc * 16, 16)
                rows[p][k, sl] = rows[p][k, sl] * w
            return carry
        lax.fori_loop(0, K, _k, 0)
        pltpu.sync_copy(rows[p], agg_s.at[dstb[p]], add=True)

        @pl.when(jj + 2 < NCHUNK)
        def _():
            _issue_small(jnp.minimum(jj + 2, NCHUNK - 1), p)

    def _body(t, carry):
        _one(2 * t, 0)
        _one(2 * t + 1, 1)
        return carry
    lax.fori_loop(0, NCHUNK // 2, _body, 0)

    plsc.subcore_barrier()
    pltpu.sync_copy(agg_s.at[pl.ds(ss_ * RPT, RPT)],
                    agg_hbm.at[cc_, pl.ds(ss_ * RPT, RPT)])


def _sc_edge(h_p, asrc_p, adst_p, src_t, dst_t, b16):
    mesh = plsc.VectorSubcoreMesh(core_axis_name="c", subcore_axis_name="s")
    e_vals, den = pl.kernel(
        _sca_body,
        out_type=(jax.ShapeDtypeStruct((NW, NCHUNK, K), _F32),
                  jax.ShapeDtypeStruct((NW, NP), _F32)),
        mesh=mesh,
        scratch_types=[
            pltpu.VMEM((NP,), _F32),
            pltpu.VMEM((NP,), _F32),
            pltpu.VMEM((NP,), _F32),
            pltpu.VMEM((16,), _F32),
            pltpu.VMEM((K,), jnp.int32),
            pltpu.VMEM((K,), jnp.int32),
            pltpu.VMEM((K,), jnp.int32),
            pltpu.VMEM((K,), jnp.int32),
            pltpu.VMEM((K,), _F32),
            pltpu.VMEM((K,), _F32),
            pltpu.SemaphoreType.DMA,
            pltpu.SemaphoreType.DMA,
            pltpu.SemaphoreType.DMA,
            pltpu.SemaphoreType.DMA,
        ],
        compiler_params=pltpu.CompilerParams(needs_layout_passes=False),
    )(asrc_p, adst_p, src_t, dst_t, b16)

    agg = pl.kernel(
        _scb_body,
        out_type=jax.ShapeDtypeStruct((2, NP, C), _F32),
        mesh=mesh,
        scratch_types=[
            pltpu.VMEM((K, C), _F32),
            pltpu.VMEM((K, C), _F32),
            pltpu.VMEM((K,), jnp.int32),
            pltpu.VMEM((K,), jnp.int32),
            pltpu.VMEM((K,), jnp.int32),
            pltpu.VMEM((K,), jnp.int32),
            pltpu.VMEM((K,), _F32),
            pltpu.VMEM((K,), _F32),
            pltpu.VMEM_SHARED((NP, C), _F32),
            pltpu.SemaphoreType.DMA,
            pltpu.SemaphoreType.DMA,
            pltpu.SemaphoreType.DMA,
            pltpu.SemaphoreType.DMA,
            pltpu.SemaphoreType.DMA,
            pltpu.SemaphoreType.DMA,
            pltpu.SemaphoreType.DMA,
            pltpu.SemaphoreType.DMA,
        ],
        compiler_params=pltpu.CompilerParams(needs_layout_passes=False),
    )(h_p, src_t, dst_t, e_vals)
    return agg, den


def _tc2_body(aggp_ref, denp_ref, sk1_ref, b1_ref, l1b_ref,
              w2_ref, as2_ref, ad2_ref, lw2_ref,
              h2_ref, sk2_ref, s_ref, d_ref, b_ref, ms, md):
    i = pl.program_id(0)
    agg = aggp_ref[0] + aggp_ref[1]
    den = jnp.sum(denp_ref[...], axis=0)
    gat = agg / (den + 1e-16)[:, None]
    h = jnp.maximum(gat + b1_ref[...] + sk1_ref[...] + l1b_ref[...], 0.0)
    h2 = jnp.dot(h, w2_ref[...], precision=_HI, preferred_element_type=_F32)
    h2_ref[...] = h2
    sk2_ref[...] = lax.dot_general(h, lw2_ref[...], (((1,), (1,)), ((), ())),
                                   precision=_HI, preferred_element_type=_F32)
    s = jnp.sum(h2 * as2_ref[...], axis=1)
    d = jnp.sum(h2 * ad2_ref[...], axis=1)
    s_ref[...] = s.reshape(1, 1, R)
    d_ref[...] = d.reshape(1, 1, R)

    @pl.when(i == 0)
    def _():
        ms[0, 0] = -1e30
        md[0, 0] = -1e30

    ms[0, 0] = jnp.maximum(ms[0, 0], jnp.max(s))
    md[0, 0] = jnp.maximum(md[0, 0], jnp.max(d))

    @pl.when(i == pl.num_programs(0) - 1)
    def _():
        _attn_bound(ms, md, b_ref)


def _dense2(aggp, denp, skip1, b1, lin1_b, W2, att_src2, att_dst2, lin2_W):
    return pl.pallas_call(
        _tc2_body,
        grid=(G,),
        in_specs=[
            pl.BlockSpec((2, R, C), lambda i: (0, i, 0)),
            pl.BlockSpec((NW, R), lambda i: (0, i)),
            pl.BlockSpec((R, C), lambda i: (i, 0)),
            pl.BlockSpec((1, C), lambda i: (0, 0)),
            pl.BlockSpec((1, C), lambda i: (0, 0)),
            pl.BlockSpec((C, C), lambda i: (0, 0)),
            pl.BlockSpec((1, C), lambda i: (0, 0)),
            pl.BlockSpec((1, C), lambda i: (0, 0)),
            pl.BlockSpec((C, C), lambda i: (0, 0)),
        ],
        out_specs=[
            pl.BlockSpec((R, C), lambda i: (i, 0)),
            pl.BlockSpec((R, C), lambda i: (i, 0)),
            pl.BlockSpec((1, 1, R), lambda i: (i, 0, 0)),
            pl.BlockSpec((1, 1, R), lambda i: (i, 0, 0)),
            pl.BlockSpec((1, C), lambda i: (0, 0)),
        ],
        out_shape=[
            jax.ShapeDtypeStruct((NP, C), _F32),
            jax.ShapeDtypeStruct((NP, C), _F32),
            jax.ShapeDtypeStruct((G, 1, R), _F32),
            jax.ShapeDtypeStruct((G, 1, R), _F32),
            jax.ShapeDtypeStruct((1, C), _F32),
        ],
        scratch_shapes=[pltpu.SMEM((1, 1), _F32), pltpu.SMEM((1, 1), _F32)],
    )(aggp, denp, skip1, b1, lin1_b, W2, att_src2, att_dst2, lin2_W)


def _tc3_body(aggp_ref, denp_ref, sk2_ref, b2_ref, l2b_ref, o_ref):
    agg = aggp_ref[0] + aggp_ref[1]
    den = jnp.sum(denp_ref[...], axis=0)
    o_ref[...] = (agg / (den + 1e-16)[:, None]
                  + b2_ref[...] + sk2_ref[...] + l2b_ref[...])


def _final(aggp, denp, skip2, b2, lin2_b):
    return pl.pallas_call(
        _tc3_body,
        grid=(G,),
        in_specs=[
            pl.BlockSpec((2, R, C), lambda i: (0, i, 0)),
            pl.BlockSpec((NW, R), lambda i: (0, i)),
            pl.BlockSpec((R, C), lambda i: (i, 0)),
            pl.BlockSpec((1, C), lambda i: (0, 0)),
            pl.BlockSpec((1, C), lambda i: (0, 0)),
        ],
        out_specs=pl.BlockSpec((R, C), lambda i: (i, 0)),
        out_shape=jax.ShapeDtypeStruct((NP, C), _F32),
    )(aggp, denp, skip2, b2, lin2_b)


def kernel(x, edge_index, W1, att_src1, att_dst1, b1, lin1_W, lin1_b,
           W2, att_src2, att_dst2, b2, lin2_W, lin2_b):
    x_p = jnp.pad(x, ((0, NP - N), (0, 0)))
    src_t = jnp.pad(edge_index[0].reshape(NW, EPT),
                    ((0, 0), (0, NCHUNK * K - EPT))).reshape(NW, NCHUNK, K)
    dst_t = jnp.pad(edge_index[1].reshape(NW, EPT),
                    ((0, 0), (0, NCHUNK * K - EPT)),
                    constant_values=N).reshape(NW, NCHUNK, K)

    h1, skip1, s1, d1, B1 = _dense1(x_p, W1, att_src1.reshape(1, C),
                                    att_dst1.reshape(1, C), lin1_W)
    agg1, den1 = _sc_edge(h1, s1.reshape(NP), d1.reshape(NP),
                          src_t, dst_t, B1[0, :16])
    h2, skip2, s2, d2, B2 = _dense2(agg1, den1, skip1, b1.reshape(1, C),
                                    lin1_b.reshape(1, C), W2,
                                    att_src2.reshape(1, C),
                                    att_dst2.reshape(1, C), lin2_W)
    agg2, den2 = _sc_edge(h2, s2.reshape(NP), d2.reshape(NP),
                          src_t, dst_t, B2[0, :16])
    out = _final(agg2, den2, skip2, b2.reshape(1, C), lin2_b.reshape(1, C))
    return out[:N]


# speedup vs baseline: 18.9741x; 1.0827x over previous
import jax
import jax.numpy as jnp
from jax import lax
from jax.experimental import pallas as pl
from jax.experimental.pallas import tpu as pltpu
from jax.experimental.pallas import tpu_sc as plsc

N = 10000
NP = 10240
E = 320000
C = 128
NW = 32
EPT = E // NW
NCHUNK = 80
K = 128
RPT = NP // 16
R = 1024
G = NP // R
_F32 = jnp.float32
_HI = lax.Precision.HIGHEST


def _attn_bound(ms, md, b_ref):
    zmax = ms[0, 0] + md[0, 0]
    b_ref[...] = jnp.full((1, C), jnp.maximum(zmax, 0.2 * zmax), _F32)


def _tc1_body(x_ref, w1_ref, as_ref, ad_ref, lw_ref,
              h_ref, sk_ref, s_ref, d_ref, b_ref, ms, md):
    i = pl.program_id(0)
    xb = x_ref[...]
    h = jnp.dot(xb, w1_ref[...], precision=_HI, preferred_element_type=_F32)
    h_ref[...] = h
    sk_ref[...] = lax.dot_general(xb, lw_ref[...], (((1,), (1,)), ((), ())),
                                  precision=_HI, preferred_element_type=_F32)
    s = jnp.sum(h * as_ref[...], axis=1)
    d = jnp.sum(h * ad_ref[...], axis=1)
    s_ref[...] = s.reshape(1, 1, R)
    d_ref[...] = d.reshape(1, 1, R)

    @pl.when(i == 0)
    def _():
        ms[0, 0] = -1e30
        md[0, 0] = -1e30

    ms[0, 0] = jnp.maximum(ms[0, 0], jnp.max(s))
    md[0, 0] = jnp.maximum(md[0, 0], jnp.max(d))

    @pl.when(i == pl.num_programs(0) - 1)
    def _():
        _attn_bound(ms, md, b_ref)


def _dense1(x_p, W1, att_src, att_dst, lin_W):
    return pl.pallas_call(
        _tc1_body,
        grid=(G,),
        in_specs=[
            pl.BlockSpec((R, C), lambda i: (i, 0)),
            pl.BlockSpec((C, C), lambda i: (0, 0)),
            pl.BlockSpec((1, C), lambda i: (0, 0)),
            pl.BlockSpec((1, C), lambda i: (0, 0)),
            pl.BlockSpec((C, C), lambda i: (0, 0)),
        ],
        out_specs=[
            pl.BlockSpec((R, C), lambda i: (i, 0)),
            pl.BlockSpec((R, C), lambda i: (i, 0)),
            pl.BlockSpec((1, 1, R), lambda i: (i, 0, 0)),
            pl.BlockSpec((1, 1, R), lambda i: (i, 0, 0)),
            pl.BlockSpec((1, C), lambda i: (0, 0)),
        ],
        out_shape=[
            jax.ShapeDtypeStruct((NP, C), _F32),
            jax.ShapeDtypeStruct((NP, C), _F32),
            jax.ShapeDtypeStruct((G, 1, R), _F32),
            jax.ShapeDtypeStruct((G, 1, R), _F32),
            jax.ShapeDtypeStruct((1, C), _F32),
        ],
        scratch_shapes=[pltpu.SMEM((1, 1), _F32), pltpu.SMEM((1, 1), _F32)],
    )(x_p, W1, att_src, att_dst, lin_W)


def _sca_body(asrc_hbm, adst_hbm, src_hbm, dst_hbm, b_hbm,
              e_hbm, den_hbm,
              asrc_v, adst_v, den_v, b_v,
              srcb0, srcb1, dstb0, dstb1, eb0, eb1,
              sem_s0, sem_s1, sem_d0, sem_d1):
    cc_ = lax.axis_index("c")
    ss_ = lax.axis_index("s")
    wid = cc_ * 16 + ss_

    pltpu.sync_copy(asrc_hbm, asrc_v)
    pltpu.sync_copy(adst_hbm, adst_v)
    pltpu.sync_copy(b_hbm, b_v)

    zv = jnp.zeros((16,), _F32)

    def _zden(i, carry):
        den_v[pl.ds(i * 16, 16)] = zv
        return carry
    lax.fori_loop(0, NP // 16, _zden, 0)

    srcb = (srcb0, srcb1)
    dstb = (dstb0, dstb1)
    eb = (eb0, eb1)
    sem_s = (sem_s0, sem_s1)
    sem_d = (sem_d0, sem_d1)

    def _issue(jj, p):
        pltpu.async_copy(src_hbm.at[wid, jj], srcb[p], sem_s[p])
        pltpu.async_copy(dst_hbm.at[wid, jj], dstb[p], sem_d[p])

    def _wait(jj, p):
        pltpu.make_async_copy(src_hbm.at[wid, jj], srcb[p], sem_s[p]).wait()
        pltpu.make_async_copy(dst_hbm.at[wid, jj], dstb[p], sem_d[p]).wait()

    _issue(0, 0)
    _issue(1, 1)
    bval = b_v[...]

    def _one(jj, p):
        _wait(jj, p)

        def _g(g, carry):
            s16 = srcb[p][pl.ds(g * 16, 16)]
            d16 = dstb[p][pl.ds(g * 16, 16)]
            z = plsc.load_gather(asrc_v, [s16]) + plsc.load_gather(adst_v, [d16])
            z = jnp.maximum(z, 0.2 * z)
            e = jnp.exp(z - bval)
            eb[p][pl.ds(g * 16, 16)] = e
            plsc.addupdate_scatter(den_v, [d16], e)
            return carry
        lax.fori_loop(0, K // 16, _g, 0)
        pltpu.sync_copy(eb[p], e_hbm.at[wid, jj])

        @pl.when(jj + 2 < NCHUNK)
        def _():
            _issue(jnp.minimum(jj + 2, NCHUNK - 1), p)

    def _body(t, carry):
        _one(2 * t, 0)
        _one(2 * t + 1, 1)
        return carry
    lax.fori_loop(0, NCHUNK // 2, _body, 0)

    pltpu.sync_copy(den_v, den_hbm.at[wid])


def _scb_body(h_hbm, src_hbm, dst_hbm, e_hbm,
              agg_hbm,
              rows0, rows1, srcb0, srcb1, srcb2, srcb3,
              dstb0, dstb1, dstb2, dstb3, eb0, eb1, eb2, eb3,
              agg_s,
              sem_r0, sem_r1,
              sem_s0, sem_s1, sem_s2, sem_s3,
              sem_d0, sem_d1, sem_d2, sem_d3,
              sem_e0, sem_e1, sem_e2, sem_e3,
              sem_w0, sem_w1):
    cc_ = lax.axis_index("c")
    ss_ = lax.axis_index("s")
    wid = cc_ * 16 + ss_

    rows = (rows0, rows1)
    srcb = (srcb0, srcb1, srcb2, srcb3)
    dstb = (dstb0, dstb1, dstb2, dstb3)
    eb = (eb0, eb1, eb2, eb3)
    sem_r = (sem_r0, sem_r1)
    sem_s = (sem_s0, sem_s1, sem_s2, sem_s3)
    sem_d = (sem_d0, sem_d1, sem_d2, sem_d3)
    sem_e = (sem_e0, sem_e1, sem_e2, sem_e3)
    sem_w = (sem_w0, sem_w1)

    zv = jnp.zeros((16,), _F32)

    def _zrow(i, carry):
        for cc in range(C // 16):
            rows0[i, pl.ds(cc * 16, 16)] = zv
        return carry
    lax.fori_loop(0, K, _zrow, 0)
    for t in range(RPT // K):
        pltpu.sync_copy(rows0, agg_s.at[pl.ds(ss_ * RPT + t * K, K)])
    plsc.subcore_barrier()

    def _issue_small(jj, p4):
        pltpu.async_copy(src_hbm.at[wid, jj], srcb[p4], sem_s[p4])
        pltpu.async_copy(dst_hbm.at[wid, jj], dstb[p4], sem_d[p4])
        pltpu.async_copy(e_hbm.at[wid, jj], eb[p4], sem_e[p4])

    def _wait_small(jj, p4):
        pltpu.make_async_copy(src_hbm.at[wid, jj], srcb[p4], sem_s[p4]).wait()
        pltpu.make_async_copy(dst_hbm.at[wid, jj], dstb[p4], sem_d[p4]).wait()
        pltpu.make_async_copy(e_hbm.at[wid, jj], eb[p4], sem_e[p4]).wait()

    def _issue_rows(p4, p2):
        pltpu.async_copy(h_hbm.at[srcb[p4]], rows[p2], sem_r[p2])

    def _wait_rows(p4, p2):
        pltpu.make_async_copy(h_hbm.at[srcb[p4]], rows[p2], sem_r[p2]).wait()

    def _wait_scatter(p4, p2):
        pltpu.make_async_copy(rows[p2], agg_s.at[dstb[p4]], sem_w[p2]).wait()

    _issue_small(0, 0)
    _issue_small(1, 1)
    _wait_small(0, 0)
    _issue_rows(0, 0)

    def _one(jj, i):
        p2 = i % 2
        p4 = i % 4
        q2 = (i + 1) % 2
        q4 = (i + 1) % 4

        @pl.when(jj >= 1)
        def _():
            _wait_scatter((i + 3) % 4, q2)

        @pl.when(jj + 1 < NCHUNK)
        def _():
            jn = jnp.minimum(jj + 1, NCHUNK - 1)
            _wait_small(jn, q4)
            _issue_rows(q4, q2)

        _wait_rows(p4, p2)

        def _k(kk, carry):
            for u in range(4):
                k = kk * 4 + u
                w = plsc.load_gather(eb[p4], [jnp.full((16,), k, jnp.int32)])
                for cc in range(C // 16):
                    sl = pl.ds(cc * 16, 16)
                    rows[p2][k, sl] = rows[p2][k, sl] * w
            return carry
        lax.fori_loop(0, K // 4, _k, 0)
        pltpu.async_copy(rows[p2], agg_s.at[dstb[p4]], sem_w[p2], add=True)

        @pl.when(jj + 2 < NCHUNK)
        def _():
            _issue_small(jnp.minimum(jj + 2, NCHUNK - 1), (i + 2) % 4)

    def _body(t, carry):
        for i in range(4):
            _one(4 * t + i, i)
        return carry
    lax.fori_loop(0, NCHUNK // 4, _body, 0)

    _wait_scatter((NCHUNK - 1) % 4, (NCHUNK - 1) % 2)
    plsc.subcore_barrier()
    pltpu.sync_copy(agg_s.at[pl.ds(ss_ * RPT, RPT)],
                    agg_hbm.at[cc_, pl.ds(ss_ * RPT, RPT)])


def _sc_edge(h_p, asrc_p, adst_p, src_t, dst_t, b16):
    mesh = plsc.VectorSubcoreMesh(core_axis_name="c", subcore_axis_name="s")
    e_vals, den = pl.kernel(
        _sca_body,
        out_type=(jax.ShapeDtypeStruct((NW, NCHUNK, K), _F32),
                  jax.ShapeDtypeStruct((NW, NP), _F32)),
        mesh=mesh,
        scratch_types=[
            pltpu.VMEM((NP,), _F32),
            pltpu.VMEM((NP,), _F32),
            pltpu.VMEM((NP,), _F32),
            pltpu.VMEM((16,), _F32),
            pltpu.VMEM((K,), jnp.int32),
            pltpu.VMEM((K,), jnp.int32),
            pltpu.VMEM((K,), jnp.int32),
            pltpu.VMEM((K,), jnp.int32),
            pltpu.VMEM((K,), _F32),
            pltpu.VMEM((K,), _F32),
            pltpu.SemaphoreType.DMA,
            pltpu.SemaphoreType.DMA,
            pltpu.SemaphoreType.DMA,
            pltpu.SemaphoreType.DMA,
        ],
        compiler_params=pltpu.CompilerParams(needs_layout_passes=False),
    )(asrc_p, adst_p, src_t, dst_t, b16)

    agg = pl.kernel(
        _scb_body,
        out_type=jax.ShapeDtypeStruct((2, NP, C), _F32),
        mesh=mesh,
        scratch_types=(
            [pltpu.VMEM((K, C), _F32)] * 2
            + [pltpu.VMEM((K,), jnp.int32)] * 8
            + [pltpu.VMEM((K,), _F32)] * 4
            + [pltpu.VMEM_SHARED((NP, C), _F32)]
            + [pltpu.SemaphoreType.DMA] * 16
        ),
        compiler_params=pltpu.CompilerParams(needs_layout_passes=False),
    )(h_p, src_t, dst_t, e_vals)
    return agg, den


def _tc2_body(aggp_ref, denp_ref, sk1_ref, b1_ref, l1b_ref,
              w2_ref, as2_ref, ad2_ref, lw2_ref,
              h2_ref, sk2_ref, s_ref, d_ref, b_ref, ms, md):
    i = pl.program_id(0)
    agg = aggp_ref[0] + aggp_ref[1]
    den = jnp.sum(denp_ref[...], axis=0)
    gat = agg / (den + 1e-16)[:, None]
    h = jnp.maximum(gat + b1_ref[...] + sk1_ref[...] + l1b_ref[...], 0.0)
    h2 = jnp.dot(h, w2_ref[...], precision=_HI, preferred_element_type=_F32)
    h2_ref[...] = h2
    sk2_ref[...] = lax.dot_general(h, lw2_ref[...], (((1,), (1,)), ((), ())),
                                   precision=_HI, preferred_element_type=_F32)
    s = jnp.sum(h2 * as2_ref[...], axis=1)
    d = jnp.sum(h2 * ad2_ref[...], axis=1)
    s_ref[...] = s.reshape(1, 1, R)
    d_ref[...] = d.reshape(1, 1, R)

    @pl.when(i == 0)
    def _():
        ms[0, 0] = -1e30
        md[0, 0] = -1e30

    ms[0, 0] = jnp.maximum(ms[0, 0], jnp.max(s))
    md[0, 0] = jnp.maximum(md[0, 0], jnp.max(d))

    @pl.when(i == pl.num_programs(0) - 1)
    def _():
        _attn_bound(ms, md, b_ref)


def _dense2(aggp, denp, skip1, b1, lin1_b, W2, att_src2, att_dst2, lin2_W):
    return pl.pallas_call(
        _tc2_body,
        grid=(G,),
        in_specs=[
            pl.BlockSpec((2, R, C), lambda i: (0, i, 0)),
            pl.BlockSpec((NW, R), lambda i: (0, i)),
            pl.BlockSpec((R, C), lambda i: (i, 0)),
            pl.BlockSpec((1, C), lambda i: (0, 0)),
            pl.BlockSpec((1, C), lambda i: (0, 0)),
            pl.BlockSpec((C, C), lambda i: (0, 0)),
            pl.BlockSpec((1, C), lambda i: (0, 0)),
            pl.BlockSpec((1, C), lambda i: (0, 0)),
            pl.BlockSpec((C, C), lambda i: (0, 0)),
        ],
        out_specs=[
            pl.BlockSpec((R, C), lambda i: (i, 0)),
            pl.BlockSpec((R, C), lambda i: (i, 0)),
            pl.BlockSpec((1, 1, R), lambda i: (i, 0, 0)),
            pl.BlockSpec((1, 1, R), lambda i: (i, 0, 0)),
            pl.BlockSpec((1, C), lambda i: (0, 0)),
        ],
        out_shape=[
            jax.ShapeDtypeStruct((NP, C), _F32),
            jax.ShapeDtypeStruct((NP, C), _F32),
            jax.ShapeDtypeStruct((G, 1, R), _F32),
            jax.ShapeDtypeStruct((G, 1, R), _F32),
            jax.ShapeDtypeStruct((1, C), _F32),
        ],
        scratch_shapes=[pltpu.SMEM((1, 1), _F32), pltpu.SMEM((1, 1), _F32)],
    )(aggp, denp, skip1, b1, lin1_b, W2, att_src2, att_dst2, lin2_W)


def _tc3_body(aggp_ref, denp_ref, sk2_ref, b2_ref, l2b_ref, o_ref):
    agg = aggp_ref[0] + aggp_ref[1]
    den = jnp.sum(denp_ref[...], axis=0)
    o_ref[...] = (agg / (den + 1e-16)[:, None]
                  + b2_ref[...] + sk2_ref[...] + l2b_ref[...])


def _final(aggp, denp, skip2, b2, lin2_b):
    return pl.pallas_call(
        _tc3_body,
        grid=(G,),
        in_specs=[
            pl.BlockSpec((2, R, C), lambda i: (0, i, 0)),
            pl.BlockSpec((NW, R), lambda i: (0, i)),
            pl.BlockSpec((R, C), lambda i: (i, 0)),
            pl.BlockSpec((1, C), lambda i: (0, 0)),
            pl.BlockSpec((1, C), lambda i: (0, 0)),
        ],
        out_specs=pl.BlockSpec((R, C), lambda i: (i, 0)),
        out_shape=jax.ShapeDtypeStruct((NP, C), _F32),
    )(aggp, denp, skip2, b2, lin2_b)


def kernel(x, edge_index, W1, att_src1, att_dst1, b1, lin1_W, lin1_b,
           W2, att_src2, att_dst2, b2, lin2_W, lin2_b):
    x_p = jnp.pad(x, ((0, NP - N), (0, 0)))
    src_t = jnp.pad(edge_index[0].reshape(NW, EPT),
                    ((0, 0), (0, NCHUNK * K - EPT))).reshape(NW, NCHUNK, K)
    dst_t = jnp.pad(edge_index[1].reshape(NW, EPT),
                    ((0, 0), (0, NCHUNK * K - EPT)),
                    constant_values=N).reshape(NW, NCHUNK, K)

    h1, skip1, s1, d1, B1 = _dense1(x_p, W1, att_src1.reshape(1, C),
                                    att_dst1.reshape(1, C), lin1_W)
    agg1, den1 = _sc_edge(h1, s1.reshape(NP), d1.reshape(NP),
                          src_t, dst_t, B1[0, :16])
    h2, skip2, s2, d2, B2 = _dense2(agg1, den1, skip1, b1.reshape(1, C),
                                    lin1_b.reshape(1, C), W2,
                                    att_src2.reshape(1, C),
                                    att_dst2.reshape(1, C), lin2_W)
    agg2, den2 = _sc_edge(h2, s2.reshape(NP), d2.reshape(NP),
                          src_t, dst_t, B2[0, :16])
    out = _final(agg2, den2, skip2, b2.reshape(1, C), lin2_b.reshape(1, C))
    return out[:N]


# speedup vs baseline: 20.1651x; 1.0628x over previous
import jax
import jax.numpy as jnp
from jax import lax
from jax.experimental import pallas as pl
from jax.experimental.pallas import tpu as pltpu
from jax.experimental.pallas import tpu_sc as plsc

N = 10000
NP = 10240
E = 320000
C = 128
NW = 32
EPT = E // NW
NCHUNK = 80
K = 128
RPT = NP // 16
R = 1024
G = NP // R
_F32 = jnp.float32
_HI = lax.Precision.HIGHEST


def _attn_bound(ms, md, b_ref):
    zmax = ms[0, 0] + md[0, 0]
    b_ref[...] = jnp.full((1, C), jnp.maximum(zmax, 0.2 * zmax), _F32)


def _tc1_body(x_ref, w1_ref, as_ref, ad_ref, lw_ref,
              h_ref, sk_ref, s_ref, d_ref, b_ref, ms, md):
    i = pl.program_id(0)
    xb = x_ref[...]
    h = jnp.dot(xb, w1_ref[...], precision=_HI, preferred_element_type=_F32)
    h_ref[...] = h
    sk_ref[...] = lax.dot_general(xb, lw_ref[...], (((1,), (1,)), ((), ())),
                                  precision=_HI, preferred_element_type=_F32)
    s = jnp.sum(h * as_ref[...], axis=1)
    d = jnp.sum(h * ad_ref[...], axis=1)
    s_ref[...] = s.reshape(1, 1, R)
    d_ref[...] = d.reshape(1, 1, R)

    @pl.when(i == 0)
    def _():
        ms[0, 0] = -1e30
        md[0, 0] = -1e30

    ms[0, 0] = jnp.maximum(ms[0, 0], jnp.max(s))
    md[0, 0] = jnp.maximum(md[0, 0], jnp.max(d))

    @pl.when(i == pl.num_programs(0) - 1)
    def _():
        _attn_bound(ms, md, b_ref)


def _dense1(x_p, W1, att_src, att_dst, lin_W):
    return pl.pallas_call(
        _tc1_body,
        grid=(G,),
        in_specs=[
            pl.BlockSpec((R, C), lambda i: (i, 0)),
            pl.BlockSpec((C, C), lambda i: (0, 0)),
            pl.BlockSpec((1, C), lambda i: (0, 0)),
            pl.BlockSpec((1, C), lambda i: (0, 0)),
            pl.BlockSpec((C, C), lambda i: (0, 0)),
        ],
        out_specs=[
            pl.BlockSpec((R, C), lambda i: (i, 0)),
            pl.BlockSpec((R, C), lambda i: (i, 0)),
            pl.BlockSpec((1, 1, R), lambda i: (i, 0, 0)),
            pl.BlockSpec((1, 1, R), lambda i: (i, 0, 0)),
            pl.BlockSpec((1, C), lambda i: (0, 0)),
        ],
        out_shape=[
            jax.ShapeDtypeStruct((NP, C), _F32),
            jax.ShapeDtypeStruct((NP, C), _F32),
            jax.ShapeDtypeStruct((G, 1, R), _F32),
            jax.ShapeDtypeStruct((G, 1, R), _F32),
            jax.ShapeDtypeStruct((1, C), _F32),
        ],
        scratch_shapes=[pltpu.SMEM((1, 1), _F32), pltpu.SMEM((1, 1), _F32)],
    )(x_p, W1, att_src, att_dst, lin_W)


def _sca_body(asrc_hbm, adst_hbm, src_hbm, dst_hbm, b_hbm,
              e_hbm, den_hbm,
              asrc_v, adst_v, den_v, b_v,
              srcb0, srcb1, dstb0, dstb1, eb0, eb1,
              sem_s0, sem_s1, sem_d0, sem_d1):
    cc_ = lax.axis_index("c")
    ss_ = lax.axis_index("s")
    wid = cc_ * 16 + ss_

    pltpu.sync_copy(asrc_hbm, asrc_v)
    pltpu.sync_copy(adst_hbm, adst_v)
    pltpu.sync_copy(b_hbm, b_v)

    zv = jnp.zeros((16,), _F32)

    def _zden(i, carry):
        den_v[pl.ds(i * 16, 16)] = zv
        return carry
    lax.fori_loop(0, NP // 16, _zden, 0)

    srcb = (srcb0, srcb1)
    dstb = (dstb0, dstb1)
    eb = (eb0, eb1)
    sem_s = (sem_s0, sem_s1)
    sem_d = (sem_d0, sem_d1)

    def _issue(jj, p):
        pltpu.async_copy(src_hbm.at[wid, jj], srcb[p], sem_s[p])
        pltpu.async_copy(dst_hbm.at[wid, jj], dstb[p], sem_d[p])

    def _wait(jj, p):
        pltpu.make_async_copy(src_hbm.at[wid, jj], srcb[p], sem_s[p]).wait()
        pltpu.make_async_copy(dst_hbm.at[wid, jj], dstb[p], sem_d[p]).wait()

    _issue(0, 0)
    _issue(1, 1)
    bval = b_v[...]

    def _one(jj, p):
        _wait(jj, p)

        def _g(g, carry):
            s16 = srcb[p][pl.ds(g * 16, 16)]
            d16 = dstb[p][pl.ds(g * 16, 16)]
            z = plsc.load_gather(asrc_v, [s16]) + plsc.load_gather(adst_v, [d16])
            z = jnp.maximum(z, 0.2 * z)
            e = jnp.exp(z - bval)
            eb[p][pl.ds(g * 16, 16)] = e
            plsc.addupdate_scatter(den_v, [d16], e)
            return carry
        lax.fori_loop(0, K // 16, _g, 0)
        pltpu.sync_copy(eb[p], e_hbm.at[wid, jj])

        @pl.when(jj + 2 < NCHUNK)
        def _():
            _issue(jnp.minimum(jj + 2, NCHUNK - 1), p)

    def _body(t, carry):
        _one(2 * t, 0)
        _one(2 * t + 1, 1)
        return carry
    lax.fori_loop(0, NCHUNK // 2, _body, 0)

    pltpu.sync_copy(den_v, den_hbm.at[wid])


def _scb_body(h_hbm, src_hbm, dst_hbm, e_hbm,
              agg_hbm,
              rows0, rows1, srcb0, srcb1, srcb2, srcb3,
              dstb0, dstb1, dstb2, dstb3, eb0, eb1, eb2, eb3,
              agg_s,
              sem_r0, sem_r1,
              sem_s0, sem_s1, sem_s2, sem_s3,
              sem_d0, sem_d1, sem_d2, sem_d3,
              sem_e0, sem_e1, sem_e2, sem_e3,
              sem_w0, sem_w1):
    cc_ = lax.axis_index("c")
    ss_ = lax.axis_index("s")
    wid = cc_ * 16 + ss_

    rows = (rows0, rows1)
    srcb = (srcb0, srcb1, srcb2, srcb3)
    dstb = (dstb0, dstb1, dstb2, dstb3)
    eb = (eb0, eb1, eb2, eb3)
    sem_r = (sem_r0, sem_r1)
    sem_s = (sem_s0, sem_s1, sem_s2, sem_s3)
    sem_d = (sem_d0, sem_d1, sem_d2, sem_d3)
    sem_e = (sem_e0, sem_e1, sem_e2, sem_e3)
    sem_w = (sem_w0, sem_w1)

    zv = jnp.zeros((16,), _F32)

    def _zrow(i, carry):
        for cc in range(C // 16):
            rows0[i, pl.ds(cc * 16, 16)] = zv
        return carry
    lax.fori_loop(0, K, _zrow, 0)
    for t in range(RPT // K):
        pltpu.sync_copy(rows0, agg_s.at[pl.ds(ss_ * RPT + t * K, K)])
    plsc.subcore_barrier()

    def _issue_small(jj, p4):
        pltpu.async_copy(src_hbm.at[wid, jj], srcb[p4], sem_s[p4])
        pltpu.async_copy(dst_hbm.at[wid, jj], dstb[p4], sem_d[p4])
        pltpu.async_copy(e_hbm.at[wid, jj], eb[p4], sem_e[p4])

    def _wait_small(jj, p4):
        pltpu.make_async_copy(src_hbm.at[wid, jj], srcb[p4], sem_s[p4]).wait()
        pltpu.make_async_copy(dst_hbm.at[wid, jj], dstb[p4], sem_d[p4]).wait()
        pltpu.make_async_copy(e_hbm.at[wid, jj], eb[p4], sem_e[p4]).wait()

    def _issue_rows(p4, p2):
        pltpu.async_copy(h_hbm.at[srcb[p4]], rows[p2], sem_r[p2])

    def _wait_rows(p4, p2):
        pltpu.make_async_copy(h_hbm.at[srcb[p4]], rows[p2], sem_r[p2]).wait()

    def _wait_scatter(p4, p2):
        pltpu.make_async_copy(rows[p2], agg_s.at[dstb[p4]], sem_w[p2]).wait()

    _issue_small(0, 0)
    _issue_small(1, 1)
    _wait_small(0, 0)
    _issue_rows(0, 0)

    def _one(jj, i):
        p2 = i % 2
        p4 = i % 4
        q2 = (i + 1) % 2
        q4 = (i + 1) % 4

        @pl.when(jj >= 1)
        def _():
            _wait_scatter((i + 3) % 4, q2)

        @pl.when(jj + 1 < NCHUNK)
        def _():
            jn = jnp.minimum(jj + 1, NCHUNK - 1)
            _wait_small(jn, q4)
            _issue_rows(q4, q2)

        _wait_rows(p4, p2)

        @plsc.parallel_loop(0, K, step=1, unroll=4)
        def _k(k):
            w = plsc.load_gather(eb[p4], [jnp.full((16,), k, jnp.int32)])
            for cc in range(C // 16):
                sl = pl.ds(cc * 16, 16)
                rows[p2][k, sl] = rows[p2][k, sl] * w
        pltpu.async_copy(rows[p2], agg_s.at[dstb[p4]], sem_w[p2], add=True)

        @pl.when(jj + 2 < NCHUNK)
        def _():
            _issue_small(jnp.minimum(jj + 2, NCHUNK - 1), (i + 2) % 4)

    def _body(t, carry):
        for i in range(4):
            _one(4 * t + i, i)
        return carry
    lax.fori_loop(0, NCHUNK // 4, _body, 0)

    _wait_scatter((NCHUNK - 1) % 4, (NCHUNK - 1) % 2)
    plsc.subcore_barrier()
    pltpu.sync_copy(agg_s.at[pl.ds(ss_ * RPT, RPT)],
                    agg_hbm.at[cc_, pl.ds(ss_ * RPT, RPT)])


def _sc_edge(h_p, asrc_p, adst_p, src_t, dst_t, b16):
    mesh = plsc.VectorSubcoreMesh(core_axis_name="c", subcore_axis_name="s")
    e_vals, den = pl.kernel(
        _sca_body,
        out_type=(jax.ShapeDtypeStruct((NW, NCHUNK, K), _F32),
                  jax.ShapeDtypeStruct((NW, NP), _F32)),
        mesh=mesh,
        scratch_types=[
            pltpu.VMEM((NP,), _F32),
            pltpu.VMEM((NP,), _F32),
            pltpu.VMEM((NP,), _F32),
            pltpu.VMEM((16,), _F32),
            pltpu.VMEM((K,), jnp.int32),
            pltpu.VMEM((K,), jnp.int32),
            pltpu.VMEM((K,), jnp.int32),
            pltpu.VMEM((K,), jnp.int32),
            pltpu.VMEM((K,), _F32),
            pltpu.VMEM((K,), _F32),
            pltpu.SemaphoreType.DMA,
            pltpu.SemaphoreType.DMA,
            pltpu.SemaphoreType.DMA,
            pltpu.SemaphoreType.DMA,
        ],
        compiler_params=pltpu.CompilerParams(needs_layout_passes=False),
    )(asrc_p, adst_p, src_t, dst_t, b16)

    agg = pl.kernel(
        _scb_body,
        out_type=jax.ShapeDtypeStruct((2, NP, C), _F32),
        mesh=mesh,
        scratch_types=(
            [pltpu.VMEM((K, C), _F32)] * 2
            + [pltpu.VMEM((K,), jnp.int32)] * 8
            + [pltpu.VMEM((K,), _F32)] * 4
            + [pltpu.VMEM_SHARED((NP, C), _F32)]
            + [pltpu.SemaphoreType.DMA] * 16
        ),
        compiler_params=pltpu.CompilerParams(needs_layout_passes=False),
    )(h_p, src_t, dst_t, e_vals)
    return agg, den


def _tc2_body(aggp_ref, denp_ref, sk1_ref, b1_ref, l1b_ref,
              w2_ref, as2_ref, ad2_ref, lw2_ref,
              h2_ref, sk2_ref, s_ref, d_ref, b_ref, ms, md):
    i = pl.program_id(0)
    agg = aggp_ref[0] + aggp_ref[1]
    den = jnp.sum(denp_ref[...], axis=0)
    gat = agg / (den + 1e-16)[:, None]
    h = jnp.maximum(gat + b1_ref[...] + sk1_ref[...] + l1b_ref[...], 0.0)
    h2 = jnp.dot(h, w2_ref[...], precision=_HI, preferred_element_type=_F32)
    h2_ref[...] = h2
    sk2_ref[...] = lax.dot_general(h, lw2_ref[...], (((1,), (1,)), ((), ())),
                                   precision=_HI, preferred_element_type=_F32)
    s = jnp.sum(h2 * as2_ref[...], axis=1)
    d = jnp.sum(h2 * ad2_ref[...], axis=1)
    s_ref[...] = s.reshape(1, 1, R)
    d_ref[...] = d.reshape(1, 1, R)

    @pl.when(i == 0)
    def _():
        ms[0, 0] = -1e30
        md[0, 0] = -1e30

    ms[0, 0] = jnp.maximum(ms[0, 0], jnp.max(s))
    md[0, 0] = jnp.maximum(md[0, 0], jnp.max(d))

    @pl.when(i == pl.num_programs(0) - 1)
    def _():
        _attn_bound(ms, md, b_ref)


def _dense2(aggp, denp, skip1, b1, lin1_b, W2, att_src2, att_dst2, lin2_W):
    return pl.pallas_call(
        _tc2_body,
        grid=(G,),
        in_specs=[
            pl.BlockSpec((2, R, C), lambda i: (0, i, 0)),
            pl.BlockSpec((NW, R), lambda i: (0, i)),
            pl.BlockSpec((R, C), lambda i: (i, 0)),
            pl.BlockSpec((1, C), lambda i: (0, 0)),
            pl.BlockSpec((1, C), lambda i: (0, 0)),
            pl.BlockSpec((C, C), lambda i: (0, 0)),
            pl.BlockSpec((1, C), lambda i: (0, 0)),
            pl.BlockSpec((1, C), lambda i: (0, 0)),
            pl.BlockSpec((C, C), lambda i: (0, 0)),
        ],
        out_specs=[
            pl.BlockSpec((R, C), lambda i: (i, 0)),
            pl.BlockSpec((R, C), lambda i: (i, 0)),
            pl.BlockSpec((1, 1, R), lambda i: (i, 0, 0)),
            pl.BlockSpec((1, 1, R), lambda i: (i, 0, 0)),
            pl.BlockSpec((1, C), lambda i: (0, 0)),
        ],
        out_shape=[
            jax.ShapeDtypeStruct((NP, C), _F32),
            jax.ShapeDtypeStruct((NP, C), _F32),
            jax.ShapeDtypeStruct((G, 1, R), _F32),
            jax.ShapeDtypeStruct((G, 1, R), _F32),
            jax.ShapeDtypeStruct((1, C), _F32),
        ],
        scratch_shapes=[pltpu.SMEM((1, 1), _F32), pltpu.SMEM((1, 1), _F32)],
    )(aggp, denp, skip1, b1, lin1_b, W2, att_src2, att_dst2, lin2_W)


def _tc3_body(aggp_ref, denp_ref, sk2_ref, b2_ref, l2b_ref, o_ref):
    agg = aggp_ref[0] + aggp_ref[1]
    den = jnp.sum(denp_ref[...], axis=0)
    o_ref[...] = (agg / (den + 1e-16)[:, None]
                  + b2_ref[...] + sk2_ref[...] + l2b_ref[...])


def _final(aggp, denp, skip2, b2, lin2_b):
    return pl.pallas_call(
        _tc3_body,
        grid=(G,),
        in_specs=[
            pl.BlockSpec((2, R, C), lambda i: (0, i, 0)),
            pl.BlockSpec((NW, R), lambda i: (0, i)),
            pl.BlockSpec((R, C), lambda i: (i, 0)),
            pl.BlockSpec((1, C), lambda i: (0, 0)),
            pl.BlockSpec((1, C), lambda i: (0, 0)),
        ],
        out_specs=pl.BlockSpec((R, C), lambda i: (i, 0)),
        out_shape=jax.ShapeDtypeStruct((NP, C), _F32),
    )(aggp, denp, skip2, b2, lin2_b)


def kernel(x, edge_index, W1, att_src1, att_dst1, b1, lin1_W, lin1_b,
           W2, att_src2, att_dst2, b2, lin2_W, lin2_b):
    x_p = jnp.pad(x, ((0, NP - N), (0, 0)))
    src_t = jnp.pad(edge_index[0].reshape(NW, EPT),
                    ((0, 0), (0, NCHUNK * K - EPT))).reshape(NW, NCHUNK, K)
    dst_t = jnp.pad(edge_index[1].reshape(NW, EPT),
                    ((0, 0), (0, NCHUNK * K - EPT)),
                    constant_values=N).reshape(NW, NCHUNK, K)

    h1, skip1, s1, d1, B1 = _dense1(x_p, W1, att_src1.reshape(1, C),
                                    att_dst1.reshape(1, C), lin1_W)
    agg1, den1 = _sc_edge(h1, s1.reshape(NP), d1.reshape(NP),
                          src_t, dst_t, B1[0, :16])
    h2, skip2, s2, d2, B2 = _dense2(agg1, den1, skip1, b1.reshape(1, C),
                                    lin1_b.reshape(1, C), W2,
                                    att_src2.reshape(1, C),
                                    att_dst2.reshape(1, C), lin2_W)
    agg2, den2 = _sc_edge(h2, s2.reshape(NP), d2.reshape(NP),
                          src_t, dst_t, B2[0, :16])
    out = _final(agg2, den2, skip2, b2.reshape(1, C), lin2_b.reshape(1, C))
    return out[:N]


# speedup vs baseline: 20.2784x; 1.0056x over previous
import jax
import jax.numpy as jnp
from jax import lax
from jax.experimental import pallas as pl
from jax.experimental.pallas import tpu as pltpu
from jax.experimental.pallas import tpu_sc as plsc

N = 10000
NP = 10240
E = 320000
C = 128
NW = 32
EPT = E // NW
NCHUNK = 160
K = 64
RPT = NP // 16
R = 1024
G = NP // R
_F32 = jnp.float32
_HI = lax.Precision.HIGHEST


def _attn_bound(ms, md, b_ref):
    zmax = ms[0, 0] + md[0, 0]
    b_ref[...] = jnp.full((1, C), jnp.maximum(zmax, 0.2 * zmax), _F32)


def _tc1_body(x_ref, w1_ref, as_ref, ad_ref, lw_ref,
              hb_ref, sk_ref, s_ref, d_ref, b_ref, ms, md):
    i = pl.program_id(0)
    xb = x_ref[...]
    h = jnp.dot(xb, w1_ref[...], precision=_HI, preferred_element_type=_F32)
    hb_ref[...] = h
    sk_ref[...] = lax.dot_general(xb, lw_ref[...], (((1,), (1,)), ((), ())),
                                  precision=_HI, preferred_element_type=_F32)
    s = jnp.sum(h * as_ref[...], axis=1)
    d = jnp.sum(h * ad_ref[...], axis=1)
    s_ref[...] = s.reshape(1, 1, R)
    d_ref[...] = d.reshape(1, 1, R)

    @pl.when(i == 0)
    def _():
        ms[0, 0] = -1e30
        md[0, 0] = -1e30

    ms[0, 0] = jnp.maximum(ms[0, 0], jnp.max(s))
    md[0, 0] = jnp.maximum(md[0, 0], jnp.max(d))

    @pl.when(i == pl.num_programs(0) - 1)
    def _():
        _attn_bound(ms, md, b_ref)


def _dense1(x_p, W1, att_src, att_dst, lin_W):
    return pl.pallas_call(
        _tc1_body,
        grid=(G,),
        in_specs=[
            pl.BlockSpec((R, C), lambda i: (i, 0)),
            pl.BlockSpec((C, C), lambda i: (0, 0)),
            pl.BlockSpec((1, C), lambda i: (0, 0)),
            pl.BlockSpec((1, C), lambda i: (0, 0)),
            pl.BlockSpec((C, C), lambda i: (0, 0)),
        ],
        out_specs=[
            pl.BlockSpec((R, C), lambda i: (i, 0)),
            pl.BlockSpec((R, C), lambda i: (i, 0)),
            pl.BlockSpec((1, 1, R), lambda i: (i, 0, 0)),
            pl.BlockSpec((1, 1, R), lambda i: (i, 0, 0)),
            pl.BlockSpec((1, C), lambda i: (0, 0)),
        ],
        out_shape=[
            jax.ShapeDtypeStruct((NP, C), _F32),
            jax.ShapeDtypeStruct((NP, C), _F32),
            jax.ShapeDtypeStruct((G, 1, R), _F32),
            jax.ShapeDtypeStruct((G, 1, R), _F32),
            jax.ShapeDtypeStruct((1, C), _F32),
        ],
        scratch_shapes=[pltpu.SMEM((1, 1), _F32), pltpu.SMEM((1, 1), _F32)],
    )(x_p, W1, att_src, att_dst, lin_W)


def _sc_body(h_hbm, asrc_hbm, adst_hbm, src_hbm, dst_hbm, b_hbm,
             agg_hbm, den_hbm,
             asrc_v, adst_v, den_v, b_v,
             rows0, rows1, srcb0, srcb1, srcb2, srcb3,
             dstb0, dstb1, dstb2, dstb3, ebuf0, ebuf1,
             agg_s,
             sem_r0, sem_r1,
             sem_s0, sem_s1, sem_s2, sem_s3,
             sem_d0, sem_d1, sem_d2, sem_d3,
             sem_w0, sem_w1):
    cc_ = lax.axis_index("c")
    ss_ = lax.axis_index("s")
    wid = cc_ * 16 + ss_

    pltpu.sync_copy(asrc_hbm, asrc_v)
    pltpu.sync_copy(adst_hbm, adst_v)
    pltpu.sync_copy(b_hbm, b_v)

    rows = (rows0, rows1)
    srcb = (srcb0, srcb1, srcb2, srcb3)
    dstb = (dstb0, dstb1, dstb2, dstb3)
    ebuf = (ebuf0, ebuf1)
    sem_r = (sem_r0, sem_r1)
    sem_s = (sem_s0, sem_s1, sem_s2, sem_s3)
    sem_d = (sem_d0, sem_d1, sem_d2, sem_d3)
    sem_w = (sem_w0, sem_w1)

    zv = jnp.zeros((16,), _F32)

    def _zrow(i, carry):
        for cc in range(C // 16):
            rows0[i, pl.ds(cc * 16, 16)] = zv
        return carry
    lax.fori_loop(0, K, _zrow, 0)
    for t in range(RPT // K):
        pltpu.sync_copy(rows0, agg_s.at[pl.ds(ss_ * RPT + t * K, K)])

    def _zden(i, carry):
        den_v[pl.ds(i * 16, 16)] = zv
        return carry
    lax.fori_loop(0, NP // 16, _zden, 0)
    plsc.subcore_barrier()

    def _issue_small(jj, p4):
        pltpu.async_copy(src_hbm.at[wid, jj], srcb[p4], sem_s[p4])
        pltpu.async_copy(dst_hbm.at[wid, jj], dstb[p4], sem_d[p4])

    def _wait_small(jj, p4):
        pltpu.make_async_copy(src_hbm.at[wid, jj], srcb[p4], sem_s[p4]).wait()
        pltpu.make_async_copy(dst_hbm.at[wid, jj], dstb[p4], sem_d[p4]).wait()

    def _issue_rows(p4, p2):
        pltpu.async_copy(h_hbm.at[srcb[p4]], rows[p2], sem_r[p2])

    def _wait_rows(p4, p2):
        pltpu.make_async_copy(h_hbm.at[srcb[p4]], rows[p2], sem_r[p2]).wait()

    def _wait_scatter(p4, p2):
        pltpu.make_async_copy(rows[p2], agg_s.at[dstb[p4]], sem_w[p2]).wait()

    _issue_small(0, 0)
    _issue_small(1, 1)
    _wait_small(0, 0)
    _issue_rows(0, 0)
    bval = b_v[...]

    def _one(jj, i):
        p2 = i % 2
        p4 = i % 4
        q2 = (i + 1) % 2
        q4 = (i + 1) % 4

        @pl.when(jj >= 1)
        def _():
            _wait_scatter((i + 3) % 4, q2)

        @pl.when(jj + 1 < NCHUNK)
        def _():
            jn = jnp.minimum(jj + 1, NCHUNK - 1)
            _wait_small(jn, q4)
            _issue_rows(q4, q2)

        @plsc.parallel_loop(0, K // 16, step=1, unroll=4)
        def _g(g):
            s16 = srcb[p4][pl.ds(g * 16, 16)]
            d16 = dstb[p4][pl.ds(g * 16, 16)]
            z = plsc.load_gather(asrc_v, [s16]) + plsc.load_gather(adst_v, [d16])
            z = jnp.maximum(z, 0.2 * z)
            e = jnp.exp(z - bval)
            ebuf[p2][pl.ds(g * 16, 16)] = e
            plsc.addupdate_scatter(den_v, [d16], e)

        _wait_rows(p4, p2)

        @plsc.parallel_loop(0, K, step=1, unroll=4)
        def _k(k):
            w = plsc.load_gather(ebuf[p2], [jnp.full((16,), k, jnp.int32)])
            for cc in range(C // 16):
                sl = pl.ds(cc * 16, 16)
                rows[p2][k, sl] = rows[p2][k, sl] * w
        pltpu.async_copy(rows[p2], agg_s.at[dstb[p4]], sem_w[p2], add=True)

        @pl.when(jj + 2 < NCHUNK)
        def _():
            _issue_small(jnp.minimum(jj + 2, NCHUNK - 1), (i + 2) % 4)

    def _body(t, carry):
        for i in range(4):
            _one(4 * t + i, i)
        return carry
    lax.fori_loop(0, NCHUNK // 4, _body, 0)

    _wait_scatter((NCHUNK - 1) % 4, (NCHUNK - 1) % 2)
    pltpu.sync_copy(den_v, den_hbm.at[wid])
    plsc.subcore_barrier()
    pltpu.sync_copy(agg_s.at[pl.ds(ss_ * RPT, RPT)],
                    agg_hbm.at[cc_, pl.ds(ss_ * RPT, RPT)])


def _sc_edge(h_p, asrc_p, adst_p, src_t, dst_t, b16):
    mesh = plsc.VectorSubcoreMesh(core_axis_name="c", subcore_axis_name="s")
    agg, den = pl.kernel(
        _sc_body,
        out_type=(jax.ShapeDtypeStruct((2, NP, C), _F32),
                  jax.ShapeDtypeStruct((NW, NP), _F32)),
        mesh=mesh,
        scratch_types=(
            [pltpu.VMEM((NP,), _F32)] * 3
            + [pltpu.VMEM((16,), _F32)]
            + [pltpu.VMEM((K, C), _F32)] * 2
            + [pltpu.VMEM((K,), jnp.int32)] * 8
            + [pltpu.VMEM((K,), _F32)] * 2
            + [pltpu.VMEM_SHARED((NP, C), _F32)]
            + [pltpu.SemaphoreType.DMA] * 12
        ),
        compiler_params=pltpu.CompilerParams(needs_layout_passes=False),
    )(h_p, asrc_p, adst_p, src_t, dst_t, b16)
    return agg, den


def _tc2_body(aggp_ref, denp_ref, sk1_ref, b1_ref, l1b_ref,
              w2_ref, as2_ref, ad2_ref, lw2_ref,
              h2b_ref, sk2_ref, s_ref, d_ref, b_ref, ms, md):
    i = pl.program_id(0)
    agg = aggp_ref[0] + aggp_ref[1]
    den = jnp.sum(denp_ref[...], axis=0)
    gat = agg / (den + 1e-16)[:, None]
    h = jnp.maximum(gat + b1_ref[...] + sk1_ref[...] + l1b_ref[...], 0.0)
    h2 = jnp.dot(h, w2_ref[...], precision=_HI, preferred_element_type=_F32)
    h2b_ref[...] = h2
    sk2_ref[...] = lax.dot_general(h, lw2_ref[...], (((1,), (1,)), ((), ())),
                                   precision=_HI, preferred_element_type=_F32)
    s = jnp.sum(h2 * as2_ref[...], axis=1)
    d = jnp.sum(h2 * ad2_ref[...], axis=1)
    s_ref[...] = s.reshape(1, 1, R)
    d_ref[...] = d.reshape(1, 1, R)

    @pl.when(i == 0)
    def _():
        ms[0, 0] = -1e30
        md[0, 0] = -1e30

    ms[0, 0] = jnp.maximum(ms[0, 0], jnp.max(s))
    md[0, 0] = jnp.maximum(md[0, 0], jnp.max(d))

    @pl.when(i == pl.num_programs(0) - 1)
    def _():
        _attn_bound(ms, md, b_ref)


def _dense2(aggp, denp, skip1, b1, lin1_b, W2, att_src2, att_dst2, lin2_W):
    return pl.pallas_call(
        _tc2_body,
        grid=(G,),
        in_specs=[
            pl.BlockSpec((2, R, C), lambda i: (0, i, 0)),
            pl.BlockSpec((NW, R), lambda i: (0, i)),
            pl.BlockSpec((R, C), lambda i: (i, 0)),
            pl.BlockSpec((1, C), lambda i: (0, 0)),
            pl.BlockSpec((1, C), lambda i: (0, 0)),
            pl.BlockSpec((C, C), lambda i: (0, 0)),
            pl.BlockSpec((1, C), lambda i: (0, 0)),
            pl.BlockSpec((1, C), lambda i: (0, 0)),
            pl.BlockSpec((C, C), lambda i: (0, 0)),
        ],
        out_specs=[
            pl.BlockSpec((R, C), lambda i: (i, 0)),
            pl.BlockSpec((R, C), lambda i: (i, 0)),
            pl.BlockSpec((1, 1, R), lambda i: (i, 0, 0)),
            pl.BlockSpec((1, 1, R), lambda i: (i, 0, 0)),
            pl.BlockSpec((1, C), lambda i: (0, 0)),
        ],
        out_shape=[
            jax.ShapeDtypeStruct((NP, C), _F32),
            jax.ShapeDtypeStruct((NP, C), _F32),
            jax.ShapeDtypeStruct((G, 1, R), _F32),
            jax.ShapeDtypeStruct((G, 1, R), _F32),
            jax.ShapeDtypeStruct((1, C), _F32),
        ],
        scratch_shapes=[pltpu.SMEM((1, 1), _F32), pltpu.SMEM((1, 1), _F32)],
    )(aggp, denp, skip1, b1, lin1_b, W2, att_src2, att_dst2, lin2_W)


def _tc3_body(aggp_ref, denp_ref, sk2_ref, b2_ref, l2b_ref, o_ref):
    agg = aggp_ref[0] + aggp_ref[1]
    den = jnp.sum(denp_ref[...], axis=0)
    o_ref[...] = (agg / (den + 1e-16)[:, None]
                  + b2_ref[...] + sk2_ref[...] + l2b_ref[...])


def _final(aggp, denp, skip2, b2, lin2_b):
    return pl.pallas_call(
        _tc3_body,
        grid=(G,),
        in_specs=[
            pl.BlockSpec((2, R, C), lambda i: (0, i, 0)),
            pl.BlockSpec((NW, R), lambda i: (0, i)),
            pl.BlockSpec((R, C), lambda i: (i, 0)),
            pl.BlockSpec((1, C), lambda i: (0, 0)),
            pl.BlockSpec((1, C), lambda i: (0, 0)),
        ],
        out_specs=pl.BlockSpec((R, C), lambda i: (i, 0)),
        out_shape=jax.ShapeDtypeStruct((NP, C), _F32),
    )(aggp, denp, skip2, b2, lin2_b)


def kernel(x, edge_index, W1, att_src1, att_dst1, b1, lin1_W, lin1_b,
           W2, att_src2, att_dst2, b2, lin2_W, lin2_b):
    x_p = jnp.pad(x, ((0, NP - N), (0, 0)))
    src_t = jnp.pad(edge_index[0].reshape(NW, EPT),
                    ((0, 0), (0, NCHUNK * K - EPT))).reshape(NW, NCHUNK, K)
    dst_t = jnp.pad(edge_index[1].reshape(NW, EPT),
                    ((0, 0), (0, NCHUNK * K - EPT)),
                    constant_values=N).reshape(NW, NCHUNK, K)

    hb1, skip1, s1, d1, B1 = _dense1(x_p, W1, att_src1.reshape(1, C),
                                     att_dst1.reshape(1, C), lin1_W)
    agg1, den1 = _sc_edge(hb1, s1.reshape(NP), d1.reshape(NP),
                          src_t, dst_t, B1[0, :16])
    hb2, skip2, s2, d2, B2 = _dense2(agg1, den1, skip1, b1.reshape(1, C),
                                     lin1_b.reshape(1, C), W2,
                                     att_src2.reshape(1, C),
                                     att_dst2.reshape(1, C), lin2_W)
    agg2, den2 = _sc_edge(hb2, s2.reshape(NP), d2.reshape(NP),
                          src_t, dst_t, B2[0, :16])
    out = _final(agg2, den2, skip2, b2.reshape(1, C), lin2_b.reshape(1, C))
    return out[:N]
